# Initial kernel scaffold; baseline (speedup 1.0000x reference)
import jax
import jax.numpy as jnp
from jax.experimental import pallas as pl


def kernel(x, edge_index, W, b, lin_W, lin_b):
    raise NotImplementedError("write your pallas kernel here")


# speedup vs baseline: 13.3152x; 13.3152x over previous
import functools

import jax
import jax.numpy as jnp
from jax import lax
from jax.experimental import pallas as pl
from jax.experimental.pallas import tpu as pltpu
from jax.experimental.pallas import tpu_sc as plsc

N = 10000
E = 320000
D = 128

NC = 2
NS = 16
NW = NC * NS
C = 128
K = 80
G = 16
NGRP = K // G
E_PAD = NW * K * C
N_PAD = 10240
RPT = N_PAD // NS

_MESH = plsc.VectorSubcoreMesh(core_axis_name="c", subcore_axis_name="s")


@functools.partial(
    pl.kernel,
    out_type=jax.ShapeDtypeStruct((NC, N_PAD), jnp.float32),
    mesh=_MESH,
    scratch_types=[
        pltpu.VMEM((K, C), jnp.int32),
        pltpu.VMEM((C,), jnp.float32),
        pltpu.VMEM((RPT,), jnp.float32),
        pltpu.VMEM_SHARED((N_PAD,), jnp.float32),
        pltpu.SemaphoreType.DMA,
    ],
)
def _sc_degree(dst_hbm, deg_out, dst_v, ones_v, zero_v, deg_sh, sem):
    cid = lax.axis_index("c")
    sid = lax.axis_index("s")
    wid = cid * NS + sid
    for i in range(C // 16):
        ones_v[pl.ds(i * 16, 16)] = jnp.ones((16,), jnp.float32)
    for i in range(RPT // 16):
        zero_v[pl.ds(i * 16, 16)] = jnp.zeros((16,), jnp.float32)
    pltpu.sync_copy(zero_v, deg_sh.at[pl.ds(sid * RPT, RPT)])
    plsc.subcore_barrier()
    pltpu.sync_copy(dst_hbm.at[wid], dst_v)

    def body(j, carry):
        pltpu.sync_copy(ones_v, deg_sh.at[dst_v.at[j]], add=True)
        return carry

    lax.fori_loop(0, K, body, 0)
    plsc.subcore_barrier()
    pltpu.sync_copy(deg_sh.at[pl.ds(sid * RPT, RPT)],
                    deg_out.at[cid, pl.ds(sid * RPT, RPT)])


@functools.partial(
    pl.kernel,
    out_type=jax.ShapeDtypeStruct((NC, N_PAD, D), jnp.float32),
    mesh=_MESH,
    scratch_types=[
        pltpu.VMEM((G, C), jnp.int32),
        pltpu.VMEM((G, C), jnp.int32),
        pltpu.VMEM((C, D), jnp.float32),
        pltpu.VMEM((C, D), jnp.float32),
        pltpu.VMEM_SHARED((N_PAD, D), jnp.float32),
        pltpu.SemaphoreType.DMA,
        pltpu.SemaphoreType.DMA,
    ],
)
def _sc_scatter(y_hbm, src_hbm, dst_hbm, acc_out,
                src_v, dst_v, rows_a, rows_b, acc_sh, sem_a, sem_b):
    cid = lax.axis_index("c")
    sid = lax.axis_index("s")
    wid = cid * NS + sid
    pltpu.sync_copy(y_hbm.at[pl.ds(sid * RPT, RPT)],
                    acc_sh.at[pl.ds(sid * RPT, RPT)])
    plsc.subcore_barrier()

    def group(g, carry):
        pltpu.sync_copy(src_hbm.at[wid, pl.ds(g * G, G)], src_v)
        pltpu.sync_copy(dst_hbm.at[wid, pl.ds(g * G, G)], dst_v)
        pend = pltpu.async_copy(y_hbm.at[src_v.at[0]], rows_a, sem_a)
        for j in range(G):
            cur = rows_a if j % 2 == 0 else rows_b
            pend.wait()
            if j + 1 < G:
                nbuf = rows_b if j % 2 == 0 else rows_a
                nsem = sem_b if j % 2 == 0 else sem_a
                pend = pltpu.async_copy(y_hbm.at[src_v.at[j + 1]], nbuf, nsem)
            pltpu.sync_copy(cur, acc_sh.at[dst_v.at[j]], add=True)
        return carry

    lax.fori_loop(0, NGRP, group, 0)
    plsc.subcore_barrier()
    pltpu.sync_copy(acc_sh.at[pl.ds(sid * RPT, RPT)],
                    acc_out.at[cid, pl.ds(sid * RPT, RPT)])


def _tc_transform_body(da_ref, db_ref, x_ref, w_ref, y_ref, dis_ref):
    deg = da_ref[...] + db_ref[...] + 1.0
    dis = lax.rsqrt(deg)
    dis_ref[...] = dis
    y_ref[...] = dis * jnp.dot(x_ref[...], w_ref[...],
                               preferred_element_type=jnp.float32)


def _tc_transform(deg_a, deg_b, x_pad, w):
    br = 1024
    grid = (N_PAD // br,)
    return pl.pallas_call(
        _tc_transform_body,
        grid=grid,
        in_specs=[
            pl.BlockSpec((br, 1), lambda i: (i, 0)),
            pl.BlockSpec((br, 1), lambda i: (i, 0)),
            pl.BlockSpec((br, D), lambda i: (i, 0)),
            pl.BlockSpec((D, D), lambda i: (0, 0)),
        ],
        out_specs=[
            pl.BlockSpec((br, D), lambda i: (i, 0)),
            pl.BlockSpec((br, 1), lambda i: (i, 0)),
        ],
        out_shape=[
            jax.ShapeDtypeStruct((N_PAD, D), jnp.float32),
            jax.ShapeDtypeStruct((N_PAD, 1), jnp.float32),
        ],
    )(deg_a, deg_b, x_pad, w)


def _tc_final_body(aa_ref, ab_ref, y_ref, dis_ref, b_ref, lw_ref, lb_ref,
                   o_ref):
    s = dis_ref[...] * (aa_ref[...] + ab_ref[...] - y_ref[...]) + b_ref[...]
    h = jnp.tanh(s)
    o_ref[...] = jnp.dot(h, lw_ref[...],
                         preferred_element_type=jnp.float32) + lb_ref[...]


def _tc_final(acc_a, acc_b, y, dis, b2, lin_w, lb2):
    br = 1024
    grid = (N_PAD // br,)
    return pl.pallas_call(
        _tc_final_body,
        grid=grid,
        in_specs=[
            pl.BlockSpec((br, D), lambda i: (i, 0)),
            pl.BlockSpec((br, D), lambda i: (i, 0)),
            pl.BlockSpec((br, D), lambda i: (i, 0)),
            pl.BlockSpec((br, 1), lambda i: (i, 0)),
            pl.BlockSpec((1, D), lambda i: (0, 0)),
            pl.BlockSpec((D, D), lambda i: (0, 0)),
            pl.BlockSpec((1, D), lambda i: (0, 0)),
        ],
        out_specs=pl.BlockSpec((br, D), lambda i: (i, 0)),
        out_shape=jax.ShapeDtypeStruct((N_PAD, D), jnp.float32),
    )(acc_a, acc_b, y, dis, b2, lin_w, lb2)


def kernel(x, edge_index, W, b, lin_W, lin_b):
    src = edge_index[0].astype(jnp.int32)
    dst = edge_index[1].astype(jnp.int32)
    pad = E_PAD - E
    src3 = jnp.concatenate([src, jnp.zeros((pad,), jnp.int32)]).reshape(NW, K, C)
    dst3 = jnp.concatenate([dst, jnp.full((pad,), N, jnp.int32)]).reshape(NW, K, C)

    deg_p = _sc_degree(dst3)
    x_pad = jnp.pad(x, ((0, N_PAD - N), (0, 0)))
    y, dis = _tc_transform(deg_p[0].reshape(N_PAD, 1),
                           deg_p[1].reshape(N_PAD, 1), x_pad, W)
    acc_p = _sc_scatter(y, src3, dst3)
    out = _tc_final(acc_p[0], acc_p[1], y, dis,
                    b.reshape(1, D), lin_W, lin_b.reshape(1, D))
    return out[:N]


# speedup vs baseline: 13.3560x; 1.0031x over previous
import functools

import jax
import jax.numpy as jnp
from jax import lax
from jax.experimental import pallas as pl
from jax.experimental.pallas import tpu as pltpu
from jax.experimental.pallas import tpu_sc as plsc

N = 10000
E = 320000
D = 128

NC = 2
NS = 16
NW = NC * NS
C = 128
K = 80
G = 16
NGRP = K // G
E_PAD = NW * K * C
N_PAD = 10240
RPT = N_PAD // NS

_MESH = plsc.VectorSubcoreMesh(core_axis_name="c", subcore_axis_name="s")


@functools.partial(
    pl.kernel,
    out_type=jax.ShapeDtypeStruct((NC, N_PAD), jnp.float32),
    mesh=_MESH,
    scratch_types=[
        pltpu.VMEM((K, C), jnp.int32),
        pltpu.VMEM((C,), jnp.float32),
        pltpu.VMEM((RPT,), jnp.float32),
        pltpu.VMEM_SHARED((N_PAD,), jnp.float32),
        pltpu.SemaphoreType.DMA,
    ],
)
def _sc_degree(dst_hbm, deg_out, dst_v, ones_v, zero_v, deg_sh, sem):
    cid = lax.axis_index("c")
    sid = lax.axis_index("s")
    wid = cid * NS + sid
    for i in range(C // 16):
        ones_v[pl.ds(i * 16, 16)] = jnp.ones((16,), jnp.float32)
    for i in range(RPT // 16):
        zero_v[pl.ds(i * 16, 16)] = jnp.zeros((16,), jnp.float32)
    pltpu.sync_copy(zero_v, deg_sh.at[pl.ds(sid * RPT, RPT)])
    plsc.subcore_barrier()
    pltpu.sync_copy(dst_hbm.at[wid], dst_v)

    def body(j, carry):
        pltpu.sync_copy(ones_v, deg_sh.at[dst_v.at[j]], add=True)
        return carry

    lax.fori_loop(0, K, body, 0)
    plsc.subcore_barrier()
    pltpu.sync_copy(deg_sh.at[pl.ds(sid * RPT, RPT)],
                    deg_out.at[cid, pl.ds(sid * RPT, RPT)])


@functools.partial(
    pl.kernel,
    out_type=jax.ShapeDtypeStruct((NC, N_PAD, D), jnp.float32),
    mesh=_MESH,
    scratch_types=[
        pltpu.VMEM((G, C), jnp.int32),
        pltpu.VMEM((G, C), jnp.int32),
        pltpu.VMEM((C, D), jnp.float32),
        pltpu.VMEM((C, D), jnp.float32),
        pltpu.VMEM_SHARED((N_PAD, D), jnp.float32),
        pltpu.SemaphoreType.DMA,
        pltpu.SemaphoreType.DMA,
    ],
)
def _sc_scatter(y_hbm, src_hbm, dst_hbm, acc_out,
                src_v, dst_v, rows_a, rows_b, acc_sh, sem_a, sem_b):
    cid = lax.axis_index("c")
    sid = lax.axis_index("s")
    wid = cid * NS + sid
    pltpu.sync_copy(y_hbm.at[pl.ds(sid * RPT, RPT)],
                    acc_sh.at[pl.ds(sid * RPT, RPT)])
    plsc.subcore_barrier()

    def group(g, carry):
        pltpu.sync_copy(src_hbm.at[wid, pl.ds(g * G, G)], src_v)
        pltpu.sync_copy(dst_hbm.at[wid, pl.ds(g * G, G)], dst_v)
        pend = pltpu.async_copy(y_hbm.at[src_v.at[0]], rows_a, sem_a)
        for j in range(G):
            cur = rows_a if j % 2 == 0 else rows_b
            pend.wait()
            if j + 1 < G:
                nbuf = rows_b if j % 2 == 0 else rows_a
                nsem = sem_b if j % 2 == 0 else sem_a
                pend = pltpu.async_copy(y_hbm.at[src_v.at[j + 1]], nbuf, nsem)
            pltpu.sync_copy(cur, acc_sh.at[dst_v.at[j]], add=True)
        return carry

    lax.fori_loop(0, NGRP, group, 0)
    plsc.subcore_barrier()
    pltpu.sync_copy(acc_sh.at[pl.ds(sid * RPT, RPT)],
                    acc_out.at[cid, pl.ds(sid * RPT, RPT)])


def _tc_transform_body(da_ref, db_ref, x_ref, w_ref, y_ref, dis_ref):
    deg = da_ref[...] + db_ref[...] + 1.0
    dis = lax.rsqrt(deg)
    dis_ref[...] = dis
    y_ref[...] = dis * jnp.dot(x_ref[...], w_ref[...],
                               preferred_element_type=jnp.float32)


def _tc_transform(deg_a, deg_b, x_pad, w):
    br = 1024
    grid = (N_PAD // br,)
    return pl.pallas_call(
        _tc_transform_body,
        grid=grid,
        in_specs=[
            pl.BlockSpec((br, 1), lambda i: (i, 0)),
            pl.BlockSpec((br, 1), lambda i: (i, 0)),
            pl.BlockSpec((br, D), lambda i: (i, 0)),
            pl.BlockSpec((D, D), lambda i: (0, 0)),
        ],
        out_specs=[
            pl.BlockSpec((br, D), lambda i: (i, 0)),
            pl.BlockSpec((br, 1), lambda i: (i, 0)),
        ],
        out_shape=[
            jax.ShapeDtypeStruct((N_PAD, D), jnp.float32),
            jax.ShapeDtypeStruct((N_PAD, 1), jnp.float32),
        ],
    )(deg_a, deg_b, x_pad, w)


def _tc_final_body(aa_ref, ab_ref, y_ref, dis_ref, b_ref, lw_ref, lb_ref,
                   o_ref):
    s = dis_ref[...] * (aa_ref[...] + ab_ref[...] - y_ref[...]) + b_ref[...]
    h = jnp.tanh(s)
    o_ref[...] = jnp.dot(h, lw_ref[...],
                         preferred_element_type=jnp.float32) + lb_ref[...]


def _tc_final(acc_a, acc_b, y, dis, b2, lin_w, lb2):
    br = 1024
    grid = (N_PAD // br,)
    return pl.pallas_call(
        _tc_final_body,
        grid=grid,
        in_specs=[
            pl.BlockSpec((br, D), lambda i: (i, 0)),
            pl.BlockSpec((br, D), lambda i: (i, 0)),
            pl.BlockSpec((br, D), lambda i: (i, 0)),
            pl.BlockSpec((br, 1), lambda i: (i, 0)),
            pl.BlockSpec((1, D), lambda i: (0, 0)),
            pl.BlockSpec((D, D), lambda i: (0, 0)),
            pl.BlockSpec((1, D), lambda i: (0, 0)),
        ],
        out_specs=pl.BlockSpec((br, D), lambda i: (i, 0)),
        out_shape=jax.ShapeDtypeStruct((N_PAD, D), jnp.float32),
    )(acc_a, acc_b, y, dis, b2, lin_w, lb2)


def kernel(x, edge_index, W, b, lin_W, lin_b):
    src = edge_index[0].astype(jnp.int32)
    dst = edge_index[1].astype(jnp.int32)
    pad = E_PAD - E
    src3 = jnp.concatenate([src, jnp.zeros((pad,), jnp.int32)]).reshape(NW, K, C)
    pad_dst = N + (jnp.arange(pad, dtype=jnp.int32) % (N_PAD - N))
    dst3 = jnp.concatenate([dst, pad_dst]).reshape(NW, K, C)

    deg_p = _sc_degree(dst3)
    x_pad = jnp.pad(x, ((0, N_PAD - N), (0, 0)))
    y, dis = _tc_transform(deg_p[0].reshape(N_PAD, 1),
                           deg_p[1].reshape(N_PAD, 1), x_pad, W)
    acc_p = _sc_scatter(y, src3, dst3)
    out = _tc_final(acc_p[0], acc_p[1], y, dis,
                    b.reshape(1, D), lin_W, lin_b.reshape(1, D))
    return out[:N]


# speedup vs baseline: 15.0253x; 1.1250x over previous
import functools

import jax
import jax.numpy as jnp
from jax import lax
from jax.experimental import pallas as pl
from jax.experimental.pallas import tpu as pltpu
from jax.experimental.pallas import tpu_sc as plsc

N = 10000
E = 320000
D = 128

NC = 2
NS = 16
NW = NC * NS
N_PAD = 10240
RPT = N_PAD // NS

C = 128
G = 16
KTOT = 160
KA = 112
KB = KTOT - KA
E_PAD = NS * KTOT * C
EA = NS * KA * C

CD = 128
KD = 80

_MESH = plsc.VectorSubcoreMesh(core_axis_name="c", subcore_axis_name="s")


@functools.partial(
    pl.kernel,
    out_type=jax.ShapeDtypeStruct((NC, N_PAD), jnp.float32),
    mesh=_MESH,
    scratch_types=[
        pltpu.VMEM((KD, CD), jnp.int32),
        pltpu.VMEM((CD,), jnp.float32),
        pltpu.VMEM((RPT,), jnp.float32),
        pltpu.VMEM_SHARED((N_PAD,), jnp.float32),
        pltpu.SemaphoreType.DMA,
    ],
)
def _sc_degree(dst_hbm, deg_out, dst_v, ones_v, zero_v, deg_sh, sem):
    cid = lax.axis_index("c")
    sid = lax.axis_index("s")
    wid = cid * NS + sid
    for i in range(CD // 16):
        ones_v[pl.ds(i * 16, 16)] = jnp.ones((16,), jnp.float32)
    for i in range(RPT // 16):
        zero_v[pl.ds(i * 16, 16)] = jnp.zeros((16,), jnp.float32)
    pltpu.sync_copy(zero_v, deg_sh.at[pl.ds(sid * RPT, RPT)])
    plsc.subcore_barrier()
    pltpu.sync_copy(dst_hbm.at[wid], dst_v)

    def body(j, carry):
        pltpu.sync_copy(ones_v, deg_sh.at[dst_v.at[j]], add=True)
        return carry

    lax.fori_loop(0, KD, body, 0)
    plsc.subcore_barrier()
    pltpu.sync_copy(deg_sh.at[pl.ds(sid * RPT, RPT)],
                    deg_out.at[cid, pl.ds(sid * RPT, RPT)])


@functools.partial(
    pl.kernel,
    out_type=jax.ShapeDtypeStruct((NC, N_PAD, D), jnp.float32),
    mesh=_MESH,
    scratch_types=[
        pltpu.VMEM((G, C), jnp.int32),
        pltpu.VMEM((G, C), jnp.int32),
        pltpu.VMEM((C, D), jnp.float32),
        pltpu.VMEM((C, D), jnp.float32),
        pltpu.VMEM_SHARED((N_PAD, D), jnp.float32),
        pltpu.SemaphoreType.DMA,
        pltpu.SemaphoreType.DMA,
    ],
)
def _sc_scatter(y_hbm, src_hbm, dst_hbm, acc_out,
                src_v, dst_v, rows_a, rows_b, acc_sh, sem_a, sem_b):
    cid = lax.axis_index("c")
    sid = lax.axis_index("s")
    wid = cid * NS + sid
    pltpu.sync_copy(y_hbm.at[pl.ds(sid * RPT, RPT)],
                    acc_sh.at[pl.ds(sid * RPT, RPT)])
    plsc.subcore_barrier()

    def group(g, carry):
        pltpu.sync_copy(src_hbm.at[wid, pl.ds(g * G, G)], src_v)
        pltpu.sync_copy(dst_hbm.at[wid, pl.ds(g * G, G)], dst_v)
        pend = pltpu.async_copy(y_hbm.at[src_v.at[0]], rows_a, sem_a)
        for j in range(G):
            cur = rows_a if j % 2 == 0 else rows_b
            pend.wait()
            if j + 1 < G:
                nbuf = rows_b if j % 2 == 0 else rows_a
                nsem = sem_b if j % 2 == 0 else sem_a
                pend = pltpu.async_copy(y_hbm.at[src_v.at[j + 1]],
                                        nbuf, nsem)
            pltpu.sync_copy(cur, acc_sh.at[dst_v.at[j]], add=True)
        return carry

    ngrp = jnp.where(cid == 0, KA // G, KB // G)
    lax.fori_loop(0, ngrp, group, 0)

    plsc.subcore_barrier()
    pltpu.sync_copy(acc_sh.at[pl.ds(sid * RPT, RPT)],
                    acc_out.at[cid, pl.ds(sid * RPT, RPT)])


def _tc_transform_body(da_ref, db_ref, x_ref, w_ref, y_ref, dis_ref):
    deg = da_ref[...] + db_ref[...] + 1.0
    dis = lax.rsqrt(deg)
    dis_ref[...] = dis
    y_ref[...] = dis * jnp.dot(x_ref[...], w_ref[...],
                               preferred_element_type=jnp.float32)


def _tc_transform(deg_a, deg_b, x_pad, w):
    br = 1024
    grid = (N_PAD // br,)
    return pl.pallas_call(
        _tc_transform_body,
        grid=grid,
        in_specs=[
            pl.BlockSpec((br, 1), lambda i: (i, 0)),
            pl.BlockSpec((br, 1), lambda i: (i, 0)),
            pl.BlockSpec((br, D), lambda i: (i, 0)),
            pl.BlockSpec((D, D), lambda i: (0, 0)),
        ],
        out_specs=[
            pl.BlockSpec((br, D), lambda i: (i, 0)),
            pl.BlockSpec((br, 1), lambda i: (i, 0)),
        ],
        out_shape=[
            jax.ShapeDtypeStruct((N_PAD, D), jnp.float32),
            jax.ShapeDtypeStruct((N_PAD, 1), jnp.float32),
        ],
    )(deg_a, deg_b, x_pad, w)


def _tc_final_body(aa_ref, ab_ref, y_ref, dis_ref, b_ref, lw_ref, lb_ref,
                   o_ref):
    s = dis_ref[...] * (aa_ref[...] + ab_ref[...] - y_ref[...]) + b_ref[...]
    h = jnp.tanh(s)
    o_ref[...] = jnp.dot(h, lw_ref[...],
                         preferred_element_type=jnp.float32) + lb_ref[...]


def _tc_final(acc_a, acc_b, y, dis, b2, lin_w, lb2):
    br = 1024
    grid = (N_PAD // br,)
    return pl.pallas_call(
        _tc_final_body,
        grid=grid,
        in_specs=[
            pl.BlockSpec((br, D), lambda i: (i, 0)),
            pl.BlockSpec((br, D), lambda i: (i, 0)),
            pl.BlockSpec((br, D), lambda i: (i, 0)),
            pl.BlockSpec((br, 1), lambda i: (i, 0)),
            pl.BlockSpec((1, D), lambda i: (0, 0)),
            pl.BlockSpec((D, D), lambda i: (0, 0)),
            pl.BlockSpec((1, D), lambda i: (0, 0)),
        ],
        out_specs=pl.BlockSpec((br, D), lambda i: (i, 0)),
        out_shape=jax.ShapeDtypeStruct((N_PAD, D), jnp.float32),
    )(acc_a, acc_b, y, dis, b2, lin_w, lb2)


def kernel(x, edge_index, W, b, lin_W, lin_b):
    src = edge_index[0].astype(jnp.int32)
    dst = edge_index[1].astype(jnp.int32)

    pad = E_PAD - E
    pad_dst = N + (jnp.arange(pad, dtype=jnp.int32) % (N_PAD - N))
    src_p = jnp.concatenate([src, jnp.zeros((pad,), jnp.int32)])
    dst_p = jnp.concatenate([dst, pad_dst])

    deg_p = _sc_degree(dst_p.reshape(NW, KD, CD))

    x_pad = jnp.pad(x, ((0, N_PAD - N), (0, 0)))
    deg_a = deg_p[0].reshape(N_PAD, 1)
    deg_b = deg_p[1].reshape(N_PAD, 1)
    y, dis = _tc_transform(deg_a, deg_b, x_pad, W)

    src_a = src_p[:EA].reshape(NS, KA, C)
    dst_a = dst_p[:EA].reshape(NS, KA, C)
    zpad = ((0, 0), (0, KA - KB), (0, 0))
    src_b = jnp.pad(src_p[EA:].reshape(NS, KB, C), zpad)
    dst_b = jnp.pad(dst_p[EA:].reshape(NS, KB, C), zpad)
    src3 = jnp.concatenate([src_a, src_b])
    dst3 = jnp.concatenate([dst_a, dst_b])
    acc_p = _sc_scatter(y, src3, dst3)

    out = _tc_final(acc_p[0], acc_p[1], y, dis,
                    b.reshape(1, D), lin_W, lin_b.reshape(1, D))
    return out[:N]


# speedup vs baseline: 15.4788x; 1.0302x over previous
import functools

import jax
import jax.numpy as jnp
from jax import lax
from jax.experimental import pallas as pl
from jax.experimental.pallas import tpu as pltpu
from jax.experimental.pallas import tpu_sc as plsc

N = 10000
E = 320000
D = 128

NC = 2
NS = 16
NW = NC * NS
N_PAD = 10240
RPT = N_PAD // NS

C = 128
G = 16
KTOT = 160
KA = 128
KB = KTOT - KA
E_PAD = NS * KTOT * C
EA = NS * KA * C

CD = 128
KD = 80

_MESH = plsc.VectorSubcoreMesh(core_axis_name="c", subcore_axis_name="s")


@functools.partial(
    pl.kernel,
    out_type=jax.ShapeDtypeStruct((NC, N_PAD), jnp.float32),
    mesh=_MESH,
    scratch_types=[
        pltpu.VMEM((KD, CD), jnp.int32),
        pltpu.VMEM((CD,), jnp.float32),
        pltpu.VMEM((RPT,), jnp.float32),
        pltpu.VMEM_SHARED((N_PAD,), jnp.float32),
        pltpu.SemaphoreType.DMA,
    ],
)
def _sc_degree(dst_hbm, deg_out, dst_v, ones_v, zero_v, deg_sh, sem):
    cid = lax.axis_index("c")
    sid = lax.axis_index("s")
    wid = cid * NS + sid
    for i in range(CD // 16):
        ones_v[pl.ds(i * 16, 16)] = jnp.ones((16,), jnp.float32)
    for i in range(RPT // 16):
        zero_v[pl.ds(i * 16, 16)] = jnp.zeros((16,), jnp.float32)
    pltpu.sync_copy(zero_v, deg_sh.at[pl.ds(sid * RPT, RPT)])
    plsc.subcore_barrier()
    pltpu.sync_copy(dst_hbm.at[wid], dst_v)

    def body(j, carry):
        pltpu.sync_copy(ones_v, deg_sh.at[dst_v.at[j]], add=True)
        return carry

    lax.fori_loop(0, KD, body, 0)
    plsc.subcore_barrier()
    pltpu.sync_copy(deg_sh.at[pl.ds(sid * RPT, RPT)],
                    deg_out.at[cid, pl.ds(sid * RPT, RPT)])


@functools.partial(
    pl.kernel,
    out_type=jax.ShapeDtypeStruct((NC, N_PAD, D), jnp.float32),
    mesh=_MESH,
    scratch_types=[
        pltpu.VMEM((G, C), jnp.int32),
        pltpu.VMEM((G, C), jnp.int32),
        pltpu.VMEM((C, D), jnp.float32),
        pltpu.VMEM((C, D), jnp.float32),
        pltpu.VMEM_SHARED((N_PAD, D), jnp.float32),
        pltpu.SemaphoreType.DMA,
        pltpu.SemaphoreType.DMA,
    ],
)
def _sc_scatter(y_hbm, src_hbm, dst_hbm, acc_out,
                src_v, dst_v, rows_a, rows_b, acc_sh, sem_a, sem_b):
    cid = lax.axis_index("c")
    sid = lax.axis_index("s")
    wid = cid * NS + sid
    pltpu.sync_copy(y_hbm.at[pl.ds(sid * RPT, RPT)],
                    acc_sh.at[pl.ds(sid * RPT, RPT)])
    plsc.subcore_barrier()

    def group(g, carry):
        pltpu.sync_copy(src_hbm.at[wid, pl.ds(g * G, G)], src_v)
        pltpu.sync_copy(dst_hbm.at[wid, pl.ds(g * G, G)], dst_v)
        pend = pltpu.async_copy(y_hbm.at[src_v.at[0]], rows_a, sem_a)
        for j in range(G):
            cur = rows_a if j % 2 == 0 else rows_b
            pend.wait()
            if j + 1 < G:
                nbuf = rows_b if j % 2 == 0 else rows_a
                nsem = sem_b if j % 2 == 0 else sem_a
                pend = pltpu.async_copy(y_hbm.at[src_v.at[j + 1]],
                                        nbuf, nsem)
            pltpu.sync_copy(cur, acc_sh.at[dst_v.at[j]], add=True)
        return carry

    ngrp = jnp.where(cid == 0, KA // G, KB // G)
    lax.fori_loop(0, ngrp, group, 0)

    plsc.subcore_barrier()
    pltpu.sync_copy(acc_sh.at[pl.ds(sid * RPT, RPT)],
                    acc_out.at[cid, pl.ds(sid * RPT, RPT)])


def _tc_transform_body(da_ref, db_ref, x_ref, w_ref, y_ref, dis_ref):
    deg = da_ref[...] + db_ref[...] + 1.0
    dis = lax.rsqrt(deg)
    dis_ref[...] = dis
    y_ref[...] = dis * jnp.dot(x_ref[...], w_ref[...],
                               preferred_element_type=jnp.float32)


def _tc_transform(deg_a, deg_b, x_pad, w):
    br = 1024
    grid = (N_PAD // br,)
    return pl.pallas_call(
        _tc_transform_body,
        grid=grid,
        in_specs=[
            pl.BlockSpec((br, 1), lambda i: (i, 0)),
            pl.BlockSpec((br, 1), lambda i: (i, 0)),
            pl.BlockSpec((br, D), lambda i: (i, 0)),
            pl.BlockSpec((D, D), lambda i: (0, 0)),
        ],
        out_specs=[
            pl.BlockSpec((br, D), lambda i: (i, 0)),
            pl.BlockSpec((br, 1), lambda i: (i, 0)),
        ],
        out_shape=[
            jax.ShapeDtypeStruct((N_PAD, D), jnp.float32),
            jax.ShapeDtypeStruct((N_PAD, 1), jnp.float32),
        ],
    )(deg_a, deg_b, x_pad, w)


def _tc_final_body(aa_ref, ab_ref, y_ref, dis_ref, b_ref, lw_ref, lb_ref,
                   o_ref):
    s = dis_ref[...] * (aa_ref[...] + ab_ref[...] - y_ref[...]) + b_ref[...]
    h = jnp.tanh(s)
    o_ref[...] = jnp.dot(h, lw_ref[...],
                         preferred_element_type=jnp.float32) + lb_ref[...]


def _tc_final(acc_a, acc_b, y, dis, b2, lin_w, lb2):
    br = 1024
    grid = (N_PAD // br,)
    return pl.pallas_call(
        _tc_final_body,
        grid=grid,
        in_specs=[
            pl.BlockSpec((br, D), lambda i: (i, 0)),
            pl.BlockSpec((br, D), lambda i: (i, 0)),
            pl.BlockSpec((br, D), lambda i: (i, 0)),
            pl.BlockSpec((br, 1), lambda i: (i, 0)),
            pl.BlockSpec((1, D), lambda i: (0, 0)),
            pl.BlockSpec((D, D), lambda i: (0, 0)),
            pl.BlockSpec((1, D), lambda i: (0, 0)),
        ],
        out_specs=pl.BlockSpec((br, D), lambda i: (i, 0)),
        out_shape=jax.ShapeDtypeStruct((N_PAD, D), jnp.float32),
    )(acc_a, acc_b, y, dis, b2, lin_w, lb2)


def kernel(x, edge_index, W, b, lin_W, lin_b):
    src = edge_index[0].astype(jnp.int32)
    dst = edge_index[1].astype(jnp.int32)

    pad = E_PAD - E
    pad_dst = N + (jnp.arange(pad, dtype=jnp.int32) % (N_PAD - N))
    src_p = jnp.concatenate([src, jnp.zeros((pad,), jnp.int32)])
    dst_p = jnp.concatenate([dst, pad_dst])

    deg_p = _sc_degree(dst_p.reshape(NW, KD, CD))

    x_pad = jnp.pad(x, ((0, N_PAD - N), (0, 0)))
    deg_a = deg_p[0].reshape(N_PAD, 1)
    deg_b = deg_p[1].reshape(N_PAD, 1)
    y, dis = _tc_transform(deg_a, deg_b, x_pad, W)

    src_a = src_p[:EA].reshape(NS, KA, C)
    dst_a = dst_p[:EA].reshape(NS, KA, C)
    zpad = ((0, 0), (0, KA - KB), (0, 0))
    src_b = jnp.pad(src_p[EA:].reshape(NS, KB, C), zpad)
    dst_b = jnp.pad(dst_p[EA:].reshape(NS, KB, C), zpad)
    src3 = jnp.concatenate([src_a, src_b])
    dst3 = jnp.concatenate([dst_a, dst_b])
    acc_p = _sc_scatter(y, src3, dst3)

    out = _tc_final(acc_p[0], acc_p[1], y, dis,
                    b.reshape(1, D), lin_W, lin_b.reshape(1, D))
    return out[:N]


# speedup vs baseline: 16.6813x; 1.0777x over previous
import functools

import jax
import jax.numpy as jnp
from jax import lax
from jax.experimental import pallas as pl
from jax.experimental.pallas import tpu as pltpu
from jax.experimental.pallas import tpu_sc as plsc

N = 10000
E = 320000
D = 128

NC = 2
NS = 16
NW = NC * NS
N_PAD = 10240
RPT = N_PAD // NS

C = 128
G = 16
KTOT = 160
KA = 144
KB = KTOT - KA
E_PAD = NS * KTOT * C
EA = NS * KA * C

CD = 128
KD = 80

_MESH = plsc.VectorSubcoreMesh(core_axis_name="c", subcore_axis_name="s")


@functools.partial(
    pl.kernel,
    out_type=jax.ShapeDtypeStruct((NC, N_PAD), jnp.float32),
    mesh=_MESH,
    scratch_types=[
        pltpu.VMEM((KD, CD), jnp.int32),
        pltpu.VMEM((CD,), jnp.float32),
        pltpu.VMEM((RPT,), jnp.float32),
        pltpu.VMEM_SHARED((N_PAD,), jnp.float32),
        pltpu.SemaphoreType.DMA,
    ],
)
def _sc_degree(dst_hbm, deg_out, dst_v, ones_v, zero_v, deg_sh, sem):
    cid = lax.axis_index("c")
    sid = lax.axis_index("s")
    wid = cid * NS + sid
    for i in range(CD // 16):
        ones_v[pl.ds(i * 16, 16)] = jnp.ones((16,), jnp.float32)
    for i in range(RPT // 16):
        zero_v[pl.ds(i * 16, 16)] = jnp.zeros((16,), jnp.float32)
    pltpu.sync_copy(zero_v, deg_sh.at[pl.ds(sid * RPT, RPT)])
    plsc.subcore_barrier()
    pltpu.sync_copy(dst_hbm.at[wid], dst_v)

    def body(j, carry):
        pltpu.sync_copy(ones_v, deg_sh.at[dst_v.at[j]], add=True)
        return carry

    lax.fori_loop(0, KD, body, 0)
    plsc.subcore_barrier()
    pltpu.sync_copy(deg_sh.at[pl.ds(sid * RPT, RPT)],
                    deg_out.at[cid, pl.ds(sid * RPT, RPT)])


@functools.partial(
    pl.kernel,
    out_type=jax.ShapeDtypeStruct((NC, N_PAD, D), jnp.float32),
    mesh=_MESH,
    scratch_types=[
        pltpu.VMEM((G, C), jnp.int32),
        pltpu.VMEM((G, C), jnp.int32),
        pltpu.VMEM((C, D), jnp.float32),
        pltpu.VMEM((C, D), jnp.float32),
        pltpu.VMEM_SHARED((N_PAD, D), jnp.float32),
        pltpu.SemaphoreType.DMA,
        pltpu.SemaphoreType.DMA,
    ],
)
def _sc_scatter(y_hbm, src_hbm, dst_hbm, acc_out,
                src_v, dst_v, rows_a, rows_b, acc_sh, sem_a, sem_b):
    cid = lax.axis_index("c")
    sid = lax.axis_index("s")
    wid = cid * NS + sid
    pltpu.sync_copy(y_hbm.at[pl.ds(sid * RPT, RPT)],
                    acc_sh.at[pl.ds(sid * RPT, RPT)])
    plsc.subcore_barrier()

    def group(g, carry):
        pltpu.sync_copy(src_hbm.at[wid, pl.ds(g * G, G)], src_v)
        pltpu.sync_copy(dst_hbm.at[wid, pl.ds(g * G, G)], dst_v)
        pend = pltpu.async_copy(y_hbm.at[src_v.at[0]], rows_a, sem_a)
        for j in range(G):
            cur = rows_a if j % 2 == 0 else rows_b
            pend.wait()
            if j + 1 < G:
                nbuf = rows_b if j % 2 == 0 else rows_a
                nsem = sem_b if j % 2 == 0 else sem_a
                pend = pltpu.async_copy(y_hbm.at[src_v.at[j + 1]],
                                        nbuf, nsem)
            pltpu.sync_copy(cur, acc_sh.at[dst_v.at[j]], add=True)
        return carry

    ngrp = jnp.where(cid == 0, KA // G, KB // G)
    lax.fori_loop(0, ngrp, group, 0)

    plsc.subcore_barrier()
    pltpu.sync_copy(acc_sh.at[pl.ds(sid * RPT, RPT)],
                    acc_out.at[cid, pl.ds(sid * RPT, RPT)])


def _tc_transform_body(da_ref, db_ref, x_ref, w_ref, y_ref, dis_ref):
    deg = da_ref[...] + db_ref[...] + 1.0
    dis = lax.rsqrt(deg)
    dis_ref[...] = dis
    y_ref[...] = dis * jnp.dot(x_ref[...], w_ref[...],
                               preferred_element_type=jnp.float32)


def _tc_transform(deg_a, deg_b, x_pad, w):
    br = 1024
    grid = (N_PAD // br,)
    return pl.pallas_call(
        _tc_transform_body,
        grid=grid,
        in_specs=[
            pl.BlockSpec((br, 1), lambda i: (i, 0)),
            pl.BlockSpec((br, 1), lambda i: (i, 0)),
            pl.BlockSpec((br, D), lambda i: (i, 0)),
            pl.BlockSpec((D, D), lambda i: (0, 0)),
        ],
        out_specs=[
            pl.BlockSpec((br, D), lambda i: (i, 0)),
            pl.BlockSpec((br, 1), lambda i: (i, 0)),
        ],
        out_shape=[
            jax.ShapeDtypeStruct((N_PAD, D), jnp.float32),
            jax.ShapeDtypeStruct((N_PAD, 1), jnp.float32),
        ],
    )(deg_a, deg_b, x_pad, w)


def _tc_final_body(aa_ref, ab_ref, y_ref, dis_ref, b_ref, lw_ref, lb_ref,
                   o_ref):
    s = dis_ref[...] * (aa_ref[...] + ab_ref[...] - y_ref[...]) + b_ref[...]
    h = jnp.tanh(s)
    o_ref[...] = jnp.dot(h, lw_ref[...],
                         preferred_element_type=jnp.float32) + lb_ref[...]


def _tc_final(acc_a, acc_b, y, dis, b2, lin_w, lb2):
    br = 1024
    grid = (N_PAD // br,)
    return pl.pallas_call(
        _tc_final_body,
        grid=grid,
        in_specs=[
            pl.BlockSpec((br, D), lambda i: (i, 0)),
            pl.BlockSpec((br, D), lambda i: (i, 0)),
            pl.BlockSpec((br, D), lambda i: (i, 0)),
            pl.BlockSpec((br, 1), lambda i: (i, 0)),
            pl.BlockSpec((1, D), lambda i: (0, 0)),
            pl.BlockSpec((D, D), lambda i: (0, 0)),
            pl.BlockSpec((1, D), lambda i: (0, 0)),
        ],
        out_specs=pl.BlockSpec((br, D), lambda i: (i, 0)),
        out_shape=jax.ShapeDtypeStruct((N_PAD, D), jnp.float32),
    )(acc_a, acc_b, y, dis, b2, lin_w, lb2)


def kernel(x, edge_index, W, b, lin_W, lin_b):
    src = edge_index[0].astype(jnp.int32)
    dst = edge_index[1].astype(jnp.int32)

    pad = E_PAD - E
    pad_dst = N + (jnp.arange(pad, dtype=jnp.int32) % (N_PAD - N))
    src_p = jnp.concatenate([src, jnp.zeros((pad,), jnp.int32)])
    dst_p = jnp.concatenate([dst, pad_dst])

    deg_p = _sc_degree(dst_p.reshape(NW, KD, CD))

    x_pad = jnp.pad(x, ((0, N_PAD - N), (0, 0)))
    deg_a = deg_p[0].reshape(N_PAD, 1)
    deg_b = deg_p[1].reshape(N_PAD, 1)
    y, dis = _tc_transform(deg_a, deg_b, x_pad, W)

    src_a = src_p[:EA].reshape(NS, KA, C)
    dst_a = dst_p[:EA].reshape(NS, KA, C)
    zpad = ((0, 0), (0, KA - KB), (0, 0))
    src_b = jnp.pad(src_p[EA:].reshape(NS, KB, C), zpad)
    dst_b = jnp.pad(dst_p[EA:].reshape(NS, KB, C), zpad)
    src3 = jnp.concatenate([src_a, src_b])
    dst3 = jnp.concatenate([dst_a, dst_b])
    acc_p = _sc_scatter(y, src3, dst3)

    out = _tc_final(acc_p[0], acc_p[1], y, dis,
                    b.reshape(1, D), lin_W, lin_b.reshape(1, D))
    return out[:N]
